# baseline (speedup 1.0000x reference)
import functools

import jax
import jax.numpy as jnp
import numpy as np
from jax import lax
from jax.experimental import pallas as pl
from jax.experimental.pallas import tpu as pltpu
from jax.experimental.pallas import tpu_sc as plsc

F32 = jnp.float32

_NC, _NS = 2, 16
_NW = _NC * _NS
_CH = 128


def _sigmoid(x):
    return 1.0 / (1.0 + jnp.exp(-x))


def _silu(x):
    return x * _sigmoid(x)


def _proj_body(h_ref, wi_ref, wj_ref, a_ref, b_ref):
    h = h_ref[...]
    a_ref[...] = jnp.dot(h, wi_ref[...], preferred_element_type=F32)
    b_ref[...] = jnp.dot(h, wj_ref[...], preferred_element_type=F32)


def _proj(h, wi, wj):
    n, d = h.shape
    blk = 2000
    return pl.pallas_call(
        _proj_body,
        grid=(n // blk,),
        in_specs=[
            pl.BlockSpec((blk, d), lambda i: (i, 0)),
            pl.BlockSpec((d, 64), lambda i: (0, 0)),
            pl.BlockSpec((d, 64), lambda i: (0, 0)),
        ],
        out_specs=[
            pl.BlockSpec((blk, 64), lambda i: (i, 0)),
            pl.BlockSpec((blk, 64), lambda i: (i, 0)),
        ],
        out_shape=[
            jax.ShapeDtypeStruct((n, 64), F32),
            jax.ShapeDtypeStruct((n, 64), F32),
        ],
    )(h, wi, wj)


def _pipe_loop(nfull, fire, wait, proc):
    for b in range(min(2, nfull)):
        fire(b, b)

    def pair(p, _):
        for b in range(2):
            cidx = 2 * p + b
            wait(cidx, b)
            proc(cidx, b)

            @pl.when(cidx + 2 < nfull)
            def _():
                fire(cidx + 2, b)
        return 0

    lax.fori_loop(0, nfull // 2, pair, 0)
    if nfull % 2:
        cidx = nfull - 1
        wait(cidx, cidx % 2)
        proc(cidx, cidx % 2)


def _gather_body(e2, e2_off, ap_hbm, bp_hbm, eidx4_hbm, g2_hbm,
                 idx_v, bufs, sg0, sg1):
    c = lax.axis_index("c")
    s = lax.axis_index("s")
    wid = s * _NC + c
    ep = e2 // _NW
    nfull = ep // _CH
    tail = ep - nfull * _CH
    base = pl.multiple_of(wid * ep, _CH)
    sg = (sg0, sg1)
    tabs = (ap_hbm, bp_hbm, ap_hbm, bp_hbm)

    for q in range(4):
        pltpu.sync_copy(eidx4_hbm.at[q, pl.ds(e2_off + base, ep)], idx_v.at[q])

    def mk(cidx, b, q, cnt):
        off = cidx * _CH
        dst = bufs.at[b, q] if cnt == _CH else bufs.at[b, q, pl.ds(0, cnt)]
        return pltpu.make_async_copy(
            tabs[q].at[idx_v.at[q, pl.ds(off, cnt)]], dst, sg[b])

    def fire(cidx, b):
        for q in range(4):
            mk(cidx, b, q, _CH).start()

    def wait(cidx, b):
        for q in range(4):
            mk(cidx, b, q, _CH).wait()

    def proc(cidx, b):
        off = pl.multiple_of(cidx * _CH, _CH)
        for q in range(4):
            pltpu.sync_copy(bufs.at[b, q],
                            g2_hbm.at[pl.ds(base + off, _CH),
                                      pl.ds(32 * q, 32)])

    _pipe_loop(nfull, fire, wait, proc)

    if tail:
        off = nfull * _CH
        for q in range(4):
            mk(nfull, 0, q, tail).start()
        for q in range(4):
            mk(nfull, 0, q, tail).wait()
        for q in range(4):
            pltpu.sync_copy(bufs.at[0, q, pl.ds(0, tail)],
                            g2_hbm.at[pl.ds(base + off, tail),
                                      pl.ds(32 * q, 32)])


def _gather(ap, bp, eidx4, e2, e2_off):
    ep = e2 // _NW
    mesh = plsc.VectorSubcoreMesh(core_axis_name="c", subcore_axis_name="s",
                                  num_cores=_NC, num_subcores=_NS)
    k = pl.kernel(
        functools.partial(_gather_body, e2, e2_off),
        mesh=mesh,
        compiler_params=pltpu.CompilerParams(use_tc_tiling_on_sc=False),
        out_type=jax.ShapeDtypeStruct((e2, 128), jnp.int32),
        scratch_types=[
            pltpu.VMEM((4, ep), jnp.int32),
            pltpu.VMEM((2, 4, _CH, 32), jnp.int32),
            pltpu.SemaphoreType.DMA,
            pltpu.SemaphoreType.DMA,
        ],
    )
    return k(ap, bp, eidx4)


def _edge_body(g_ref, ft_ref,
               w1g_ref, eb1_ref, ew2_ref, eb2_ref,
               gw1_ref, gb1_ref, gw2_ref, gb2_ref, sel_ref, out_ref):
    g = g_ref[...]
    lo = lax.bitcast_convert_type(g << 16, F32)
    hi = lax.bitcast_convert_type(g & jnp.int32(-65536), F32)
    x = jnp.concatenate([lo[:, 0:32] + lo[:, 32:64],
                         hi[:, 0:32] + hi[:, 32:64],
                         lo[:, 64:96] + lo[:, 96:128],
                         hi[:, 64:96] + hi[:, 96:128]], axis=1)
    ft = ft_ref[...]
    pre = (x
           + jax.lax.dot_general(ft, w1g_ref[...], (((0,), (0,)), ((), ())),
                                 preferred_element_type=F32)
           + eb1_ref[...])
    h = _silu(pre)
    e = _silu(jnp.dot(h, ew2_ref[...], preferred_element_type=F32) + eb2_ref[...])
    g1 = (jax.lax.dot_general(ft, gw1_ref[...], (((0,), (0,)), ((), ())),
                              preferred_element_type=F32)
          + gb1_ref[...])
    gh = _silu(g1)
    glogit = jnp.dot(gh, gw2_ref[...], preferred_element_type=F32) + gb2_ref[...]
    gexp = jnp.dot(_sigmoid(glogit), sel_ref[...], preferred_element_type=F32,
                   precision=lax.Precision.HIGHEST)
    out_ref[...] = e * gexp


def _edge(g, ft, ft_off, w1g, eb1, ew2, eb2, gw1, gb1, gw2, gb2, sel):
    e = g.shape[0]
    blk = 3200
    off_blocks = ft_off // blk
    wspec = lambda shape: pl.BlockSpec(shape, lambda i: tuple(0 for _ in shape))
    return pl.pallas_call(
        _edge_body,
        grid=(e // blk,),
        in_specs=[
            pl.BlockSpec((blk, 128), lambda i: (i, 0)),
            pl.BlockSpec((20, blk), lambda i: (0, i + off_blocks)),
            wspec((20, 128)), wspec((1, 128)), wspec((128, 128)), wspec((1, 128)),
            wspec((20, 64)), wspec((1, 64)), wspec((64, 2)), wspec((1, 2)),
            wspec((2, 128)),
        ],
        out_specs=pl.BlockSpec((blk, 128), lambda i: (i, 0)),
        out_shape=jax.ShapeDtypeStruct((e, 128), F32),
    )(g, ft, w1g, eb1, ew2, eb2, gw1, gb1, gw2, gb2, sel)


def _scatter_body(e2, e2_off, n, msg_hbm, eidx4_hbm, zero_hbm, out_hbm,
                  idx_v, idx_t, rows_v, rows_t, agg_sh, sr0, sr1, si0, si1):
    c = lax.axis_index("c")
    s = lax.axis_index("s")
    wid = s * _NC + c
    ep = e2 // _NW
    nfull = ep // _CH
    tail = ep - nfull * _CH
    base = pl.multiple_of(wid * ep, _CH)
    npart = n // _NS
    sr = (sr0, sr1)
    si = (si0, si1)

    pltpu.sync_copy(zero_hbm.at[pl.ds(s * npart, npart)],
                    agg_sh.at[pl.ds(s * npart, npart)])
    plsc.subcore_barrier()

    def fire(cidx, b):
        off = pl.multiple_of(cidx * _CH, _CH)
        for par in range(2):
            pltpu.async_copy(
                msg_hbm.at[pl.ds(base + off, _CH), pl.ds(64 * par, 64)],
                rows_v.at[b, par], sr[b])
            pltpu.async_copy(
                eidx4_hbm.at[2 * par, pl.ds(e2_off + base + off, _CH)],
                idx_v.at[b, par], si[b])

    def wait(cidx, b):
        off = pl.multiple_of(cidx * _CH, _CH)
        for par in range(2):
            pltpu.make_async_copy(
                msg_hbm.at[pl.ds(base + off, _CH), pl.ds(64 * par, 64)],
                rows_v.at[b, par], sr[b]).wait()
            pltpu.make_async_copy(
                eidx4_hbm.at[2 * par, pl.ds(e2_off + base + off, _CH)],
                idx_v.at[b, par], si[b]).wait()

    def proc(cidx, b):
        for par in range(2):
            pltpu.sync_copy(rows_v.at[b, par],
                            agg_sh.at[idx_v.at[b, par]], add=True)

    _pipe_loop(nfull, fire, wait, proc)

    if tail:
        off = nfull * _CH
        for par in range(2):
            pltpu.sync_copy(
                msg_hbm.at[pl.ds(base + off, tail), pl.ds(64 * par, 64)],
                rows_t.at[par])
            pltpu.sync_copy(
                eidx4_hbm.at[2 * par, pl.ds(e2_off + base + off, tail)],
                idx_t.at[par])
            pltpu.sync_copy(rows_t.at[par], agg_sh.at[idx_t.at[par]], add=True)

    plsc.subcore_barrier()
    pltpu.sync_copy(agg_sh.at[pl.ds(s * npart, npart)],
                    out_hbm.at[pl.ds(c * n + s * npart, npart), pl.ds(0, 64)])


def _scatter(msg, eidx4, e2_off, zero):
    e2 = msg.shape[0]
    n = zero.shape[0]
    ep = e2 // _NW
    tail = ep - (ep // _CH) * _CH
    mesh = plsc.VectorSubcoreMesh(core_axis_name="c", subcore_axis_name="s",
                                  num_cores=_NC, num_subcores=_NS)
    k = pl.kernel(
        functools.partial(_scatter_body, e2, e2_off, n),
        mesh=mesh,
        compiler_params=pltpu.CompilerParams(use_tc_tiling_on_sc=False),
        out_type=jax.ShapeDtypeStruct((_NC * n, 128), F32),
        scratch_types=[
            pltpu.VMEM((2, 2, _CH), jnp.int32),
            pltpu.VMEM((2, max(tail, 8)), jnp.int32),
            pltpu.VMEM((2, 2, _CH, 64), F32),
            pltpu.VMEM((2, max(tail, 8), 64), F32),
            pltpu.VMEM_SHARED((n, 64), F32),
            pltpu.SemaphoreType.DMA,
            pltpu.SemaphoreType.DMA,
            pltpu.SemaphoreType.DMA,
            pltpu.SemaphoreType.DMA,
        ],
    )
    return k(msg, eidx4, zero)


def _node_body(na, h_ref, *refs):
    (w1a_ref, w1b_ref, nb1_ref, w2_ref, nb2_ref, g_ref, b_ref,
     out_ref) = refs[na:]
    h = h_ref[...]
    atot = refs[0][0] + refs[0][1]
    for r in refs[1:na]:
        atot = atot + r[0] + r[1]
    agg = atot[:, 0:64]
    m1 = (jnp.dot(h, w1a_ref[...], preferred_element_type=F32)
          + jnp.dot(agg, w1b_ref[...], preferred_element_type=F32)
          + nb1_ref[...])
    hm = _silu(m1)
    m = jnp.dot(hm, w2_ref[...], preferred_element_type=F32) + nb2_ref[...]
    y = h + m
    mu = jnp.mean(y, axis=-1, keepdims=True)
    yc = y - mu
    var = jnp.mean(yc * yc, axis=-1, keepdims=True)
    out_ref[...] = yc * lax.rsqrt(var + 1e-5) * g_ref[...] + b_ref[...]


def _node(h, aggs, w1a, w1b, nb1, w2, nb2, g, b):
    n, d = h.shape
    blk = 2000
    wspec = lambda shape: pl.BlockSpec(shape, lambda i: tuple(0 for _ in shape))
    return pl.pallas_call(
        functools.partial(_node_body, len(aggs)),
        grid=(n // blk,),
        in_specs=[
            pl.BlockSpec((blk, d), lambda i: (i, 0)),
            *[pl.BlockSpec((2, blk, 128), lambda i: (0, i, 0)) for _ in aggs],
            wspec((d, 2 * d)), wspec((64, 2 * d)), wspec((1, 2 * d)),
            wspec((2 * d, d)), wspec((1, d)),
            wspec((1, d)), wspec((1, d)),
        ],
        out_specs=pl.BlockSpec((blk, d), lambda i: (i, 0)),
        out_shape=jax.ShapeDtypeStruct((n, d), F32),
    )(h, *aggs, w1a, w1b, nb1, w2, nb2, g, b)


def kernel(H, edge_index, dist2, delta, edge_struct,
           eW1, eb1, eW2, eb2, gW1, gb1, gW2, gb2,
           nW1, nb1, nW2, nb2, ln_g, ln_b):
    bz, n, d = H.shape
    e = edge_index.shape[1]
    d_struct = edge_struct.shape[-1]
    assert bz == 1 and e % _NW == 0 and d == 128

    h0 = H.reshape(n, d)
    e2 = e // 2
    ft = jnp.concatenate([dist2.reshape(1, e), delta.reshape(1, e),
                          edge_struct.reshape(e, d_struct).T], axis=0)
    ft2 = jnp.concatenate([ft[:, 0::2], ft[:, 1::2]], axis=0)
    eidx4 = jnp.stack([edge_index[0, 0::2], edge_index[1, 0::2],
                       edge_index[0, 1::2], edge_index[1, 1::2]])

    sg = np.concatenate([np.arange(0, 64, 2), np.arange(1, 64, 2)])

    def bd(w):
        z = jnp.zeros_like(w)
        return jnp.concatenate([jnp.concatenate([w, z], 1),
                                jnp.concatenate([z, w], 1)], 0)

    w1g2 = bd(eW1[2 * d:][:, sg])
    eb1_2 = jnp.tile(eb1[sg], 2).reshape(1, 2 * 64)
    ew2bd = bd(eW2[sg][:, sg])
    eb2_2 = jnp.tile(eb2[sg], 2).reshape(1, 2 * 64)
    gw1bd = bd(gW1)
    gb1_2 = jnp.tile(gb1, 2).reshape(1, 2 * 32)
    gw2bd = bd(gW2)
    gb2_2 = jnp.tile(gb2, 2).reshape(1, 2)
    sel = bd(jnp.ones((1, 64), F32))

    a, bm = _proj(h0, eW1[0:d], eW1[d:2 * d])
    ap = lax.bitcast_convert_type(
        a.astype(jnp.bfloat16).reshape(n, 32, 2), jnp.int32)
    bp = lax.bitcast_convert_type(
        bm.astype(jnp.bfloat16).reshape(n, 32, 2), jnp.int32)

    ns = 5
    es2 = e2 // ns
    zero = jnp.zeros((n, 64), F32)
    aggs = []
    for kk in range(ns):
        gk = _gather(ap, bp, eidx4, es2, kk * es2)
        ek = _edge(gk, ft2, kk * es2, w1g2, eb1_2, ew2bd, eb2_2,
                   gw1bd, gb1_2, gw2bd, gb2_2, sel)
        aggs.append(_scatter(ek, eidx4, kk * es2, zero).reshape(_NC, n, 128))
    out = _node(h0, aggs,
                nW1[0:d], nW1[d:][sg], nb1.reshape(1, -1),
                nW2, nb2.reshape(1, -1),
                ln_g.reshape(1, -1), ln_b.reshape(1, -1))
    return out.reshape(bz, n, d)


# speedup vs baseline: 2.7787x; 2.7787x over previous
import functools

import jax
import jax.numpy as jnp
import numpy as np
from jax import lax
from jax.experimental import pallas as pl
from jax.experimental.pallas import tpu as pltpu
from jax.experimental.pallas import tpu_sc as plsc

F32 = jnp.float32

_NC, _NS = 2, 16
_NW = _NC * _NS
_CH = 128


def _sigmoid(x):
    return 1.0 / (1.0 + jnp.exp(-x))


def _silu(x):
    return x * _sigmoid(x)


def _proj_body(h_ref, wi_ref, wj_ref, a_ref, b_ref):
    h = h_ref[...]
    a_ref[...] = jnp.dot(h, wi_ref[...], preferred_element_type=F32)
    b_ref[...] = jnp.dot(h, wj_ref[...], preferred_element_type=F32)


def _proj(h, wi, wj):
    n, d = h.shape
    blk = 2000
    return pl.pallas_call(
        _proj_body,
        grid=(n // blk,),
        in_specs=[
            pl.BlockSpec((blk, d), lambda i: (i, 0)),
            pl.BlockSpec((d, 64), lambda i: (0, 0)),
            pl.BlockSpec((d, 64), lambda i: (0, 0)),
        ],
        out_specs=[
            pl.BlockSpec((blk, 64), lambda i: (i, 0)),
            pl.BlockSpec((blk, 64), lambda i: (i, 0)),
        ],
        out_shape=[
            jax.ShapeDtypeStruct((n, 64), F32),
            jax.ShapeDtypeStruct((n, 64), F32),
        ],
    )(h, wi, wj)


def _pipe_loop(nfull, fire, wait, proc):
    for b in range(min(2, nfull)):
        fire(b, b)

    def pair(p, _):
        for b in range(2):
            cidx = 2 * p + b
            wait(cidx, b)
            proc(cidx, b)

            @pl.when(cidx + 2 < nfull)
            def _():
                fire(cidx + 2, b)
        return 0

    lax.fori_loop(0, nfull // 2, pair, 0)
    if nfull % 2:
        cidx = nfull - 1
        wait(cidx, cidx % 2)
        proc(cidx, cidx % 2)


def _gather_body(e2, e2_off, ap_hbm, bp_hbm, eidx4_hbm, g2_hbm,
                 idx_v, bufs, sg0, sg1):
    c = lax.axis_index("c")
    s = lax.axis_index("s")
    wid = s * _NC + c
    ep = e2 // _NW
    nfull = ep // _CH
    tail = ep - nfull * _CH
    base = pl.multiple_of(wid * ep, _CH)
    sg = (sg0, sg1)
    tabs = (ap_hbm, bp_hbm, ap_hbm, bp_hbm)

    for q, row in enumerate((0, 2, 1, 3)):
        pltpu.sync_copy(eidx4_hbm.at[row, pl.ds(e2_off + base, ep)],
                        idx_v.at[q])

    def mk(cidx, b, q, cnt):
        off = cidx * _CH
        dst = bufs.at[b, q] if cnt == _CH else bufs.at[b, q, pl.ds(0, cnt)]
        return pltpu.make_async_copy(
            tabs[q].at[idx_v.at[q, pl.ds(off, cnt)]], dst, sg[b])

    def fire(cidx, b):
        for q in range(4):
            mk(cidx, b, q, _CH).start()

    def wait(cidx, b):
        for q in range(4):
            mk(cidx, b, q, _CH).wait()

    def proc(cidx, b):
        off = pl.multiple_of(cidx * _CH, _CH)
        for q in range(4):
            pltpu.sync_copy(bufs.at[b, q],
                            g2_hbm.at[pl.ds(base + off, _CH),
                                      pl.ds(32 * q, 32)])

    _pipe_loop(nfull, fire, wait, proc)

    if tail:
        off = nfull * _CH
        for q in range(4):
            mk(nfull, 0, q, tail).start()
        for q in range(4):
            mk(nfull, 0, q, tail).wait()
        for q in range(4):
            pltpu.sync_copy(bufs.at[0, q, pl.ds(0, tail)],
                            g2_hbm.at[pl.ds(base + off, tail),
                                      pl.ds(32 * q, 32)])


def _gather(ap, bp, eidx4, e2, e2_off):
    ep = e2 // _NW
    mesh = plsc.VectorSubcoreMesh(core_axis_name="c", subcore_axis_name="s",
                                  num_cores=_NC, num_subcores=_NS)
    k = pl.kernel(
        functools.partial(_gather_body, e2, e2_off),
        mesh=mesh,
        compiler_params=pltpu.CompilerParams(use_tc_tiling_on_sc=False),
        out_type=jax.ShapeDtypeStruct((e2, 128), jnp.int32),
        scratch_types=[
            pltpu.VMEM((4, ep), jnp.int32),
            pltpu.VMEM((2, 4, _CH, 32), jnp.int32),
            pltpu.SemaphoreType.DMA,
            pltpu.SemaphoreType.DMA,
        ],
    )
    return k(ap, bp, eidx4)


def _edge_body(g_ref, ft_ref,
               w1g_ref, eb1_ref, ew2_ref, eb2_ref,
               gw1_ref, gb1_ref, gw2_ref, gb2_ref, sel_ref, out_ref):
    g = g_ref[...]
    lo = lax.bitcast_convert_type(g << 16, F32)
    hi = lax.bitcast_convert_type(g & jnp.int32(-65536), F32)
    x = jnp.concatenate([lo[:, 0:32] + lo[:, 32:64],
                         hi[:, 0:32] + hi[:, 32:64],
                         lo[:, 64:96] + lo[:, 96:128],
                         hi[:, 64:96] + hi[:, 96:128]], axis=1)
    ft = ft_ref[...]
    pre = (x
           + jax.lax.dot_general(ft, w1g_ref[...], (((0,), (0,)), ((), ())),
                                 preferred_element_type=F32)
           + eb1_ref[...])
    h = _silu(pre)
    e = _silu(jnp.dot(h, ew2_ref[...], preferred_element_type=F32) + eb2_ref[...])
    g1 = (jax.lax.dot_general(ft, gw1_ref[...], (((0,), (0,)), ((), ())),
                              preferred_element_type=F32)
          + gb1_ref[...])
    gh = _silu(g1)
    glogit = jnp.dot(gh, gw2_ref[...], preferred_element_type=F32) + gb2_ref[...]
    gexp = jnp.dot(_sigmoid(glogit), sel_ref[...], preferred_element_type=F32,
                   precision=lax.Precision.HIGHEST)
    out_ref[...] = e * gexp


def _edge(g, ft, ft_off, w1g, eb1, ew2, eb2, gw1, gb1, gw2, gb2, sel):
    e = g.shape[0]
    blk = 3200
    off_blocks = ft_off // blk
    wspec = lambda shape: pl.BlockSpec(shape, lambda i: tuple(0 for _ in shape))
    return pl.pallas_call(
        _edge_body,
        grid=(e // blk,),
        in_specs=[
            pl.BlockSpec((blk, 128), lambda i: (i, 0)),
            pl.BlockSpec((20, blk), lambda i: (0, i + off_blocks)),
            wspec((20, 128)), wspec((1, 128)), wspec((128, 128)), wspec((1, 128)),
            wspec((20, 64)), wspec((1, 64)), wspec((64, 2)), wspec((1, 2)),
            wspec((2, 128)),
        ],
        out_specs=pl.BlockSpec((blk, 128), lambda i: (i, 0)),
        out_shape=jax.ShapeDtypeStruct((e, 128), F32),
    )(g, ft, w1g, eb1, ew2, eb2, gw1, gb1, gw2, gb2, sel)


def _scatter_body(e2, e2_off, n, msg_hbm, eidx4_hbm, zero_hbm, out_hbm,
                  idx_v, idx_t, rows_v, rows_t, agg_sh, sr0, sr1, si0, si1):
    c = lax.axis_index("c")
    s = lax.axis_index("s")
    wid = s * _NC + c
    ep = e2 // _NW
    nfull = ep // _CH
    tail = ep - nfull * _CH
    base = pl.multiple_of(wid * ep, _CH)
    npart = n // _NS
    sr = (sr0, sr1)
    si = (si0, si1)

    pltpu.sync_copy(zero_hbm.at[pl.ds(s * npart, npart)],
                    agg_sh.at[pl.ds(s * npart, npart)])
    plsc.subcore_barrier()

    def fire(cidx, b):
        off = pl.multiple_of(cidx * _CH, _CH)
        for par in range(2):
            pltpu.async_copy(
                msg_hbm.at[pl.ds(base + off, _CH), pl.ds(64 * par, 64)],
                rows_v.at[b, par], sr[b])
            pltpu.async_copy(
                eidx4_hbm.at[par, pl.ds(e2_off + base + off, _CH)],
                idx_v.at[b, par], si[b])

    def wait(cidx, b):
        off = pl.multiple_of(cidx * _CH, _CH)
        for par in range(2):
            pltpu.make_async_copy(
                msg_hbm.at[pl.ds(base + off, _CH), pl.ds(64 * par, 64)],
                rows_v.at[b, par], sr[b]).wait()
            pltpu.make_async_copy(
                eidx4_hbm.at[par, pl.ds(e2_off + base + off, _CH)],
                idx_v.at[b, par], si[b]).wait()

    def proc(cidx, b):
        for par in range(2):
            pltpu.sync_copy(rows_v.at[b, par],
                            agg_sh.at[idx_v.at[b, par]], add=True)

    _pipe_loop(nfull, fire, wait, proc)

    if tail:
        off = nfull * _CH
        for par in range(2):
            pltpu.sync_copy(
                msg_hbm.at[pl.ds(base + off, tail), pl.ds(64 * par, 64)],
                rows_t.at[par])
            pltpu.sync_copy(
                eidx4_hbm.at[par, pl.ds(e2_off + base + off, tail)],
                idx_t.at[par])
            pltpu.sync_copy(rows_t.at[par], agg_sh.at[idx_t.at[par]], add=True)

    plsc.subcore_barrier()
    pltpu.sync_copy(agg_sh.at[pl.ds(s * npart, npart)],
                    out_hbm.at[pl.ds(c * n + s * npart, npart), pl.ds(0, 64)])


def _scatter(msg, eidx4, e2_off, zero):
    e2 = msg.shape[0]
    n = zero.shape[0]
    ep = e2 // _NW
    tail = ep - (ep // _CH) * _CH
    mesh = plsc.VectorSubcoreMesh(core_axis_name="c", subcore_axis_name="s",
                                  num_cores=_NC, num_subcores=_NS)
    k = pl.kernel(
        functools.partial(_scatter_body, e2, e2_off, n),
        mesh=mesh,
        compiler_params=pltpu.CompilerParams(use_tc_tiling_on_sc=False),
        out_type=jax.ShapeDtypeStruct((_NC * n, 128), F32),
        scratch_types=[
            pltpu.VMEM((2, 2, _CH), jnp.int32),
            pltpu.VMEM((2, max(tail, 8)), jnp.int32),
            pltpu.VMEM((2, 2, _CH, 64), F32),
            pltpu.VMEM((2, max(tail, 8), 64), F32),
            pltpu.VMEM_SHARED((n, 64), F32),
            pltpu.SemaphoreType.DMA,
            pltpu.SemaphoreType.DMA,
            pltpu.SemaphoreType.DMA,
            pltpu.SemaphoreType.DMA,
        ],
    )
    return k(msg, eidx4, zero)


def _node_body(na, h_ref, *refs):
    (w1a_ref, w1b_ref, nb1_ref, w2_ref, nb2_ref, g_ref, b_ref,
     out_ref) = refs[na:]
    h = h_ref[...]
    atot = refs[0][0] + refs[0][1]
    for r in refs[1:na]:
        atot = atot + r[0] + r[1]
    agg = atot[:, 0:64]
    m1 = (jnp.dot(h, w1a_ref[...], preferred_element_type=F32)
          + jnp.dot(agg, w1b_ref[...], preferred_element_type=F32)
          + nb1_ref[...])
    hm = _silu(m1)
    m = jnp.dot(hm, w2_ref[...], preferred_element_type=F32) + nb2_ref[...]
    y = h + m
    mu = jnp.mean(y, axis=-1, keepdims=True)
    yc = y - mu
    var = jnp.mean(yc * yc, axis=-1, keepdims=True)
    out_ref[...] = yc * lax.rsqrt(var + 1e-5) * g_ref[...] + b_ref[...]


def _node(h, aggs, w1a, w1b, nb1, w2, nb2, g, b):
    n, d = h.shape
    blk = 2000
    wspec = lambda shape: pl.BlockSpec(shape, lambda i: tuple(0 for _ in shape))
    return pl.pallas_call(
        functools.partial(_node_body, len(aggs)),
        grid=(n // blk,),
        in_specs=[
            pl.BlockSpec((blk, d), lambda i: (i, 0)),
            *[pl.BlockSpec((2, blk, 128), lambda i: (0, i, 0)) for _ in aggs],
            wspec((d, 2 * d)), wspec((64, 2 * d)), wspec((1, 2 * d)),
            wspec((2 * d, d)), wspec((1, d)),
            wspec((1, d)), wspec((1, d)),
        ],
        out_specs=pl.BlockSpec((blk, d), lambda i: (i, 0)),
        out_shape=jax.ShapeDtypeStruct((n, d), F32),
    )(h, *aggs, w1a, w1b, nb1, w2, nb2, g, b)


def kernel(H, edge_index, dist2, delta, edge_struct,
           eW1, eb1, eW2, eb2, gW1, gb1, gW2, gb2,
           nW1, nb1, nW2, nb2, ln_g, ln_b):
    bz, n, d = H.shape
    e = edge_index.shape[1]
    d_struct = edge_struct.shape[-1]
    assert bz == 1 and e % _NW == 0 and d == 128

    h0 = H.reshape(n, d)
    e2 = e // 2
    ft = jnp.concatenate([dist2.reshape(1, e), delta.reshape(1, e),
                          edge_struct.reshape(e, d_struct).T], axis=0)
    ft2 = ft.reshape(2 * 10, e2)
    eidx4 = edge_index.reshape(4, e2)

    sg = np.concatenate([np.arange(0, 64, 2), np.arange(1, 64, 2)])

    def bd(w):
        z = jnp.zeros_like(w)
        return jnp.concatenate([jnp.concatenate([w, z], 1),
                                jnp.concatenate([z, w], 1)], 0)

    def bdi(w):
        z = jnp.zeros_like(w)
        return jnp.stack([jnp.concatenate([w, z], 1),
                          jnp.concatenate([z, w], 1)],
                         axis=1).reshape(2 * w.shape[0], 2 * w.shape[1])

    w1g2 = bdi(eW1[2 * d:][:, sg])
    eb1_2 = jnp.tile(eb1[sg], 2).reshape(1, 2 * 64)
    ew2bd = bd(eW2[sg][:, sg])
    eb2_2 = jnp.tile(eb2[sg], 2).reshape(1, 2 * 64)
    gw1bd = bdi(gW1)
    gb1_2 = jnp.tile(gb1, 2).reshape(1, 2 * 32)
    gw2bd = bd(gW2)
    gb2_2 = jnp.tile(gb2, 2).reshape(1, 2)
    sel = bd(jnp.ones((1, 64), F32))

    a, bm = _proj(h0, eW1[0:d], eW1[d:2 * d])
    ap = lax.bitcast_convert_type(
        a.astype(jnp.bfloat16).reshape(n, 32, 2), jnp.int32)
    bp = lax.bitcast_convert_type(
        bm.astype(jnp.bfloat16).reshape(n, 32, 2), jnp.int32)

    ns = 5
    es2 = e2 // ns
    zero = jnp.zeros((n, 64), F32)
    aggs = []
    for kk in range(ns):
        gk = _gather(ap, bp, eidx4, es2, kk * es2)
        ek = _edge(gk, ft2, kk * es2, w1g2, eb1_2, ew2bd, eb2_2,
                   gw1bd, gb1_2, gw2bd, gb2_2, sel)
        aggs.append(_scatter(ek, eidx4, kk * es2, zero).reshape(_NC, n, 128))
    out = _node(h0, aggs,
                nW1[0:d], nW1[d:][sg], nb1.reshape(1, -1),
                nW2, nb2.reshape(1, -1),
                ln_g.reshape(1, -1), ln_b.reshape(1, -1))
    return out.reshape(bz, n, d)


# speedup vs baseline: 3.1296x; 1.1263x over previous
import functools

import jax
import jax.numpy as jnp
import numpy as np
from jax import lax
from jax.experimental import pallas as pl
from jax.experimental.pallas import tpu as pltpu
from jax.experimental.pallas import tpu_sc as plsc

F32 = jnp.float32

_NC, _NS = 2, 16
_NW = _NC * _NS
_CH = 128


def _sigmoid(x):
    return 1.0 / (1.0 + jnp.exp(-x))


def _silu(x):
    return x * _sigmoid(x)


def _pack_bf16_pair(x_lo, x_hi):
    def rnd(v):
        u = lax.bitcast_convert_type(v, jnp.int32)
        return (u + 0x7fff + ((u >> 16) & 1)) & jnp.int32(-65536)
    return (lax.shift_right_logical(rnd(x_lo), 16)
            | (rnd(x_hi) & jnp.int32(-65536)))


def _proj_body(h_ref, wi_ref, wj_ref, a_ref, b_ref):
    h = h_ref[...]
    a = jnp.dot(h, wi_ref[...], preferred_element_type=F32)
    b = jnp.dot(h, wj_ref[...], preferred_element_type=F32)
    a_ref[...] = _pack_bf16_pair(a[:, 0:32], a[:, 32:64])
    b_ref[...] = _pack_bf16_pair(b[:, 0:32], b[:, 32:64])


def _proj(h, wi, wj):
    n, d = h.shape
    blk = 2000
    return pl.pallas_call(
        _proj_body,
        grid=(n // blk,),
        in_specs=[
            pl.BlockSpec((blk, d), lambda i: (i, 0)),
            pl.BlockSpec((d, 64), lambda i: (0, 0)),
            pl.BlockSpec((d, 64), lambda i: (0, 0)),
        ],
        out_specs=[
            pl.BlockSpec((blk, 32), lambda i: (i, 0)),
            pl.BlockSpec((blk, 32), lambda i: (i, 0)),
        ],
        out_shape=[
            jax.ShapeDtypeStruct((n, 32), jnp.int32),
            jax.ShapeDtypeStruct((n, 32), jnp.int32),
        ],
    )(h, wi, wj)


def _pipe_loop(nfull, fire, wait, proc):
    for b in range(min(2, nfull)):
        fire(b, b)

    def pair(p, _):
        for b in range(2):
            cidx = 2 * p + b
            wait(cidx, b)
            proc(cidx, b)

            @pl.when(cidx + 2 < nfull)
            def _():
                fire(cidx + 2, b)
        return 0

    lax.fori_loop(0, nfull // 2, pair, 0)
    if nfull % 2:
        cidx = nfull - 1
        wait(cidx, cidx % 2)
        proc(cidx, cidx % 2)


def _gather_body(e2, e2_off, ap_hbm, bp_hbm, eidx4_hbm, g2_hbm,
                 idx_v, bufs, sg0, sg1):
    c = lax.axis_index("c")
    s = lax.axis_index("s")
    wid = s * _NC + c
    ep = e2 // _NW
    nfull = ep // _CH
    tail = ep - nfull * _CH
    base = pl.multiple_of(wid * ep, _CH)
    sg = (sg0, sg1)
    tabs = (ap_hbm, bp_hbm, ap_hbm, bp_hbm)

    for q, row in enumerate((0, 2, 1, 3)):
        pltpu.sync_copy(eidx4_hbm.at[row, pl.ds(e2_off + base, ep)],
                        idx_v.at[q])

    def mk(cidx, b, q, cnt):
        off = cidx * _CH
        dst = bufs.at[b, q] if cnt == _CH else bufs.at[b, q, pl.ds(0, cnt)]
        return pltpu.make_async_copy(
            tabs[q].at[idx_v.at[q, pl.ds(off, cnt)]], dst, sg[b])

    def fire(cidx, b):
        for q in range(4):
            mk(cidx, b, q, _CH).start()

    def wait(cidx, b):
        for q in range(4):
            mk(cidx, b, q, _CH).wait()

    def proc(cidx, b):
        off = pl.multiple_of(cidx * _CH, _CH)
        for q in range(4):
            pltpu.sync_copy(bufs.at[b, q],
                            g2_hbm.at[pl.ds(base + off, _CH),
                                      pl.ds(32 * q, 32)])

    _pipe_loop(nfull, fire, wait, proc)

    if tail:
        off = nfull * _CH
        for q in range(4):
            mk(nfull, 0, q, tail).start()
        for q in range(4):
            mk(nfull, 0, q, tail).wait()
        for q in range(4):
            pltpu.sync_copy(bufs.at[0, q, pl.ds(0, tail)],
                            g2_hbm.at[pl.ds(base + off, tail),
                                      pl.ds(32 * q, 32)])


def _gather(ap, bp, eidx4, e2, e2_off):
    ep = e2 // _NW
    mesh = plsc.VectorSubcoreMesh(core_axis_name="c", subcore_axis_name="s",
                                  num_cores=_NC, num_subcores=_NS)
    k = pl.kernel(
        functools.partial(_gather_body, e2, e2_off),
        mesh=mesh,
        compiler_params=pltpu.CompilerParams(use_tc_tiling_on_sc=False),
        out_type=jax.ShapeDtypeStruct((e2, 128), jnp.int32),
        scratch_types=[
            pltpu.VMEM((4, ep), jnp.int32),
            pltpu.VMEM((2, 4, _CH, 32), jnp.int32),
            pltpu.SemaphoreType.DMA,
            pltpu.SemaphoreType.DMA,
        ],
    )
    return k(ap, bp, eidx4)


def _edge_body(g_ref, ft_ref,
               w1g_ref, eb1_ref, ew2_ref, eb2_ref,
               gw1_ref, gb1_ref, gw2_ref, gb2_ref, sel_ref, out_ref):
    g = g_ref[...]
    lo = lax.bitcast_convert_type(g << 16, F32)
    hi = lax.bitcast_convert_type(g & jnp.int32(-65536), F32)
    x = jnp.concatenate([lo[:, 0:32] + lo[:, 32:64],
                         hi[:, 0:32] + hi[:, 32:64],
                         lo[:, 64:96] + lo[:, 96:128],
                         hi[:, 64:96] + hi[:, 96:128]], axis=1)
    ft = ft_ref[...]
    pre = (x
           + jax.lax.dot_general(ft, w1g_ref[...], (((0,), (0,)), ((), ())),
                                 preferred_element_type=F32)
           + eb1_ref[...])
    h = _silu(pre)
    e = _silu(jnp.dot(h, ew2_ref[...], preferred_element_type=F32) + eb2_ref[...])
    g1 = (jax.lax.dot_general(ft, gw1_ref[...], (((0,), (0,)), ((), ())),
                              preferred_element_type=F32)
          + gb1_ref[...])
    gh = _silu(g1)
    glogit = jnp.dot(gh, gw2_ref[...], preferred_element_type=F32) + gb2_ref[...]
    gexp = jnp.dot(_sigmoid(glogit), sel_ref[...], preferred_element_type=F32,
                   precision=lax.Precision.HIGHEST)
    out_ref[...] = e * gexp


def _edge(g, ft, ft_off, w1g, eb1, ew2, eb2, gw1, gb1, gw2, gb2, sel):
    e = g.shape[0]
    blk = 3200
    off_blocks = ft_off // blk
    wspec = lambda shape: pl.BlockSpec(shape, lambda i: tuple(0 for _ in shape))
    return pl.pallas_call(
        _edge_body,
        grid=(e // blk,),
        in_specs=[
            pl.BlockSpec((blk, 128), lambda i: (i, 0)),
            pl.BlockSpec((20, blk), lambda i: (0, i + off_blocks)),
            wspec((20, 128)), wspec((1, 128)), wspec((128, 128)), wspec((1, 128)),
            wspec((20, 64)), wspec((1, 64)), wspec((64, 2)), wspec((1, 2)),
            wspec((2, 128)),
        ],
        out_specs=pl.BlockSpec((blk, 128), lambda i: (i, 0)),
        out_shape=jax.ShapeDtypeStruct((e, 128), F32),
    )(g, ft, w1g, eb1, ew2, eb2, gw1, gb1, gw2, gb2, sel)


def _scatter_body(e2, e2_off, n, msg_hbm, eidx4_hbm, zero_hbm, out_hbm,
                  idx_v, idx_t, rows_v, rows_t, agg_sh, sr0, sr1, si0, si1):
    c = lax.axis_index("c")
    s = lax.axis_index("s")
    wid = s * _NC + c
    ep = e2 // _NW
    nfull = ep // _CH
    tail = ep - nfull * _CH
    base = pl.multiple_of(wid * ep, _CH)
    npart = n // _NS
    sr = (sr0, sr1)
    si = (si0, si1)

    pltpu.sync_copy(zero_hbm.at[pl.ds(s * npart, npart)],
                    agg_sh.at[pl.ds(s * npart, npart)])
    plsc.subcore_barrier()

    def fire(cidx, b):
        off = pl.multiple_of(cidx * _CH, _CH)
        for par in range(2):
            pltpu.async_copy(
                msg_hbm.at[pl.ds(base + off, _CH), pl.ds(64 * par, 64)],
                rows_v.at[b, par], sr[b])
            pltpu.async_copy(
                eidx4_hbm.at[par, pl.ds(e2_off + base + off, _CH)],
                idx_v.at[b, par], si[b])

    def wait(cidx, b):
        off = pl.multiple_of(cidx * _CH, _CH)
        for par in range(2):
            pltpu.make_async_copy(
                msg_hbm.at[pl.ds(base + off, _CH), pl.ds(64 * par, 64)],
                rows_v.at[b, par], sr[b]).wait()
            pltpu.make_async_copy(
                eidx4_hbm.at[par, pl.ds(e2_off + base + off, _CH)],
                idx_v.at[b, par], si[b]).wait()

    def proc(cidx, b):
        for par in range(2):
            pltpu.sync_copy(rows_v.at[b, par],
                            agg_sh.at[idx_v.at[b, par]], add=True)

    _pipe_loop(nfull, fire, wait, proc)

    if tail:
        off = nfull * _CH
        for par in range(2):
            pltpu.sync_copy(
                msg_hbm.at[pl.ds(base + off, tail), pl.ds(64 * par, 64)],
                rows_t.at[par])
            pltpu.sync_copy(
                eidx4_hbm.at[par, pl.ds(e2_off + base + off, tail)],
                idx_t.at[par])
            pltpu.sync_copy(rows_t.at[par], agg_sh.at[idx_t.at[par]], add=True)

    plsc.subcore_barrier()
    pltpu.sync_copy(agg_sh.at[pl.ds(s * npart, npart)],
                    out_hbm.at[pl.ds(c * n + s * npart, npart), pl.ds(0, 64)])


def _scatter(msg, eidx4, e2_off, zero):
    e2 = msg.shape[0]
    n = zero.shape[0]
    ep = e2 // _NW
    tail = ep - (ep // _CH) * _CH
    mesh = plsc.VectorSubcoreMesh(core_axis_name="c", subcore_axis_name="s",
                                  num_cores=_NC, num_subcores=_NS)
    k = pl.kernel(
        functools.partial(_scatter_body, e2, e2_off, n),
        mesh=mesh,
        compiler_params=pltpu.CompilerParams(use_tc_tiling_on_sc=False),
        out_type=jax.ShapeDtypeStruct((_NC * n, 128), F32),
        scratch_types=[
            pltpu.VMEM((2, 2, _CH), jnp.int32),
            pltpu.VMEM((2, max(tail, 8)), jnp.int32),
            pltpu.VMEM((2, 2, _CH, 64), F32),
            pltpu.VMEM((2, max(tail, 8), 64), F32),
            pltpu.VMEM_SHARED((n, 64), F32),
            pltpu.SemaphoreType.DMA,
            pltpu.SemaphoreType.DMA,
            pltpu.SemaphoreType.DMA,
            pltpu.SemaphoreType.DMA,
        ],
    )
    return k(msg, eidx4, zero)


def _node_body(na, h_ref, *refs):
    (w1a_ref, w1b_ref, nb1_ref, w2_ref, nb2_ref, g_ref, b_ref,
     out_ref) = refs[na:]
    h = h_ref[...]
    atot = refs[0][0] + refs[0][1]
    for r in refs[1:na]:
        atot = atot + r[0] + r[1]
    agg = atot[:, 0:64]
    m1 = (jnp.dot(h, w1a_ref[...], preferred_element_type=F32)
          + jnp.dot(agg, w1b_ref[...], preferred_element_type=F32)
          + nb1_ref[...])
    hm = _silu(m1)
    m = jnp.dot(hm, w2_ref[...], preferred_element_type=F32) + nb2_ref[...]
    y = h + m
    mu = jnp.mean(y, axis=-1, keepdims=True)
    yc = y - mu
    var = jnp.mean(yc * yc, axis=-1, keepdims=True)
    out_ref[...] = yc * lax.rsqrt(var + 1e-5) * g_ref[...] + b_ref[...]


def _node(h, aggs, w1a, w1b, nb1, w2, nb2, g, b):
    n, d = h.shape
    blk = 2000
    wspec = lambda shape: pl.BlockSpec(shape, lambda i: tuple(0 for _ in shape))
    return pl.pallas_call(
        functools.partial(_node_body, len(aggs)),
        grid=(n // blk,),
        in_specs=[
            pl.BlockSpec((blk, d), lambda i: (i, 0)),
            *[pl.BlockSpec((2, blk, 128), lambda i: (0, i, 0)) for _ in aggs],
            wspec((d, 2 * d)), wspec((64, 2 * d)), wspec((1, 2 * d)),
            wspec((2 * d, d)), wspec((1, d)),
            wspec((1, d)), wspec((1, d)),
        ],
        out_specs=pl.BlockSpec((blk, d), lambda i: (i, 0)),
        out_shape=jax.ShapeDtypeStruct((n, d), F32),
    )(h, *aggs, w1a, w1b, nb1, w2, nb2, g, b)


def kernel(H, edge_index, dist2, delta, edge_struct,
           eW1, eb1, eW2, eb2, gW1, gb1, gW2, gb2,
           nW1, nb1, nW2, nb2, ln_g, ln_b):
    bz, n, d = H.shape
    e = edge_index.shape[1]
    d_struct = edge_struct.shape[-1]
    assert bz == 1 and e % _NW == 0 and d == 128

    h0 = H.reshape(n, d)
    e2 = e // 2
    ft = jnp.concatenate([dist2.reshape(1, e), delta.reshape(1, e),
                          edge_struct.reshape(e, d_struct).T], axis=0)
    ft2 = ft.reshape(2 * 10, e2)
    eidx4 = edge_index.reshape(4, e2)

    sg = np.arange(64)

    def bd(w):
        z = jnp.zeros_like(w)
        return jnp.concatenate([jnp.concatenate([w, z], 1),
                                jnp.concatenate([z, w], 1)], 0)

    def bdi(w):
        z = jnp.zeros_like(w)
        return jnp.stack([jnp.concatenate([w, z], 1),
                          jnp.concatenate([z, w], 1)],
                         axis=1).reshape(2 * w.shape[0], 2 * w.shape[1])

    w1g2 = bdi(eW1[2 * d:][:, sg])
    eb1_2 = jnp.tile(eb1[sg], 2).reshape(1, 2 * 64)
    ew2bd = bd(eW2[sg][:, sg])
    eb2_2 = jnp.tile(eb2[sg], 2).reshape(1, 2 * 64)
    gw1bd = bdi(gW1)
    gb1_2 = jnp.tile(gb1, 2).reshape(1, 2 * 32)
    gw2bd = bd(gW2)
    gb2_2 = jnp.tile(gb2, 2).reshape(1, 2)
    sel = bd(jnp.ones((1, 64), F32))

    ap, bp = _proj(h0, eW1[0:d], eW1[d:2 * d])

    ns = 5
    es2 = e2 // ns
    zero = jnp.zeros((n, 64), F32)
    aggs = []
    for kk in range(ns):
        gk = _gather(ap, bp, eidx4, es2, kk * es2)
        ek = _edge(gk, ft2, kk * es2, w1g2, eb1_2, ew2bd, eb2_2,
                   gw1bd, gb1_2, gw2bd, gb2_2, sel)
        aggs.append(_scatter(ek, eidx4, kk * es2, zero).reshape(_NC, n, 128))
    out = _node(h0, aggs,
                nW1[0:d], nW1[d:][sg], nb1.reshape(1, -1),
                nW2, nb2.reshape(1, -1),
                ln_g.reshape(1, -1), ln_b.reshape(1, -1))
    return out.reshape(bz, n, d)
